# baseline (speedup 1.0000x reference)
import functools

import jax
import jax.numpy as jnp
from jax import lax
from jax.experimental import pallas as pl
from jax.experimental.pallas import tpu as pltpu
from jax.experimental.pallas import tpu_sc as plsc

N_FIELDS = 26
VOCAB = 100000
EMBED = 32
BATCH = 16384

NC, NS, L = 2, 16, 16
NW = NC * NS
TOTAL = BATCH * N_FIELDS
PER_W = TOTAL // NW
TCH = 832
ECH = TCH * EMBED
N_CH = PER_W // TCH
NBUF = 2

_mesh = plsc.VectorSubcoreMesh(
    core_axis_name="c", subcore_axis_name="s", num_cores=NC, num_subcores=NS
)


@functools.partial(
    pl.kernel,
    out_type=jax.ShapeDtypeStruct((TOTAL * EMBED,), jnp.float32),
    mesh=_mesh,
    compiler_params=pltpu.CompilerParams(
        use_tc_tiling_on_sc=False, needs_layout_passes=False
    ),
    scratch_types=[
        pltpu.VMEM((PER_W,), jnp.int32),
        [pltpu.VMEM((ECH,), jnp.int32)] * NBUF,
        [pltpu.VMEM((ECH,), jnp.float32)] * NBUF,
        [pltpu.SemaphoreType.DMA] * NBUF,
        [pltpu.SemaphoreType.DMA] * NBUF,
    ],
)
def _sc_gather(tt_hbm, tokens_hbm, out_hbm, tok_v, idx_vs, rows_vs, g_sems, o_sems):
    wid = lax.axis_index("s") * NC + lax.axis_index("c")
    base = wid * PER_W

    pltpu.sync_copy(tokens_hbm.at[pl.ds(base, PER_W)], tok_v)

    def build_idx(cc, k):
        def body(t, carry):
            iota = lax.iota(jnp.int32, L)
            e_lo = iota * VOCAB
            e_hi = e_lo + L * VOCAB
            p = cc * TCH + t
            tok = plsc.load_gather(tok_v, [jnp.broadcast_to(p, (L,))])
            f = lax.rem(base + p, N_FIELDS)
            s0 = tok + f * (EMBED * VOCAB)
            idx_vs[k][pl.ds(t * EMBED, L)] = s0 + e_lo
            idx_vs[k][pl.ds(t * EMBED + L, L)] = s0 + e_hi
            return carry

        lax.fori_loop(0, TCH, body, 0)

    def gather(cc, k):
        return pltpu.async_copy(
            tt_hbm.at[idx_vs[k]], rows_vs[k], g_sems[k]
        )

    def write_out(cc, k):
        return pltpu.async_copy(
            rows_vs[k],
            out_hbm.at[pl.ds((base + cc * TCH) * EMBED, ECH)],
            o_sems[k],
        )

    g_h = [None] * N_CH
    o_h = [None] * N_CH
    build_idx(0, 0)
    g_h[0] = gather(0, 0)
    for cc in range(N_CH):
        k = cc % NBUF
        nk = (cc + 1) % NBUF
        if cc + 1 < N_CH:
            if cc + 1 >= NBUF:
                o_h[cc + 1 - NBUF].wait()
            build_idx(cc + 1, nk)
            g_h[cc + 1] = gather(cc + 1, nk)
        g_h[cc].wait()
        o_h[cc] = write_out(cc, k)
    o_h[N_CH - 2].wait()
    o_h[N_CH - 1].wait()


def kernel(tokens, tables):
    f = tables.shape[0]
    d = tables.shape[-1]
    tt1d = tables.transpose(0, 2, 1).reshape(-1)
    tok_flat = tokens.reshape(-1).astype(jnp.int32)
    out = _sc_gather(tt1d, tok_flat)
    return out.reshape(tokens.shape[0], f, d)


# speedup vs baseline: 1.0024x; 1.0024x over previous
import functools

import jax
import jax.numpy as jnp
from jax import lax
from jax.experimental import pallas as pl
from jax.experimental.pallas import tpu as pltpu
from jax.experimental.pallas import tpu_sc as plsc

N_FIELDS = 26
VOCAB = 100000
EMBED = 32
BATCH = 16384

NC, NS, L = 2, 16, 16
NW = NC * NS
TOTAL = BATCH * N_FIELDS
PER_W = TOTAL // NW
TCH = 512
ECH = TCH * EMBED
N_CH = PER_W // TCH
NBUF = 2

_mesh = plsc.VectorSubcoreMesh(
    core_axis_name="c", subcore_axis_name="s", num_cores=NC, num_subcores=NS
)


@functools.partial(
    pl.kernel,
    out_type=jax.ShapeDtypeStruct((TOTAL * EMBED,), jnp.float32),
    mesh=_mesh,
    compiler_params=pltpu.CompilerParams(
        use_tc_tiling_on_sc=False, needs_layout_passes=False
    ),
    scratch_types=[
        pltpu.VMEM((PER_W,), jnp.int32),
        [pltpu.VMEM((ECH,), jnp.int32)] * NBUF,
        [pltpu.VMEM((ECH,), jnp.float32)] * NBUF,
        [pltpu.SemaphoreType.DMA] * NBUF,
        [pltpu.SemaphoreType.DMA] * NBUF,
    ],
)
def _sc_gather(tt_hbm, tokens_hbm, out_hbm, tok_v, idx_vs, rows_vs, g_sems, o_sems):
    wid = lax.axis_index("s") * NC + lax.axis_index("c")
    base = wid * PER_W

    pltpu.sync_copy(tokens_hbm.at[pl.ds(base, PER_W)], tok_v)

    def build_idx(cc, k):
        def body(t, carry):
            iota = lax.iota(jnp.int32, L)
            e_lo = iota * VOCAB
            e_hi = e_lo + L * VOCAB
            p = cc * TCH + t
            tok = plsc.load_gather(tok_v, [jnp.broadcast_to(p, (L,))])
            f = lax.rem(base + p, N_FIELDS)
            s0 = tok + f * (EMBED * VOCAB)
            idx_vs[k][pl.ds(t * EMBED, L)] = s0 + e_lo
            idx_vs[k][pl.ds(t * EMBED + L, L)] = s0 + e_hi
            return carry

        lax.fori_loop(0, TCH, body, 0)

    def gather(cc, k):
        return pltpu.async_copy(
            tt_hbm.at[idx_vs[k]], rows_vs[k], g_sems[k]
        )

    def write_out(cc, k):
        return pltpu.async_copy(
            rows_vs[k],
            out_hbm.at[pl.ds((base + cc * TCH) * EMBED, ECH)],
            o_sems[k],
        )

    g_h = [None] * N_CH
    o_h = [None] * N_CH
    build_idx(0, 0)
    g_h[0] = gather(0, 0)
    for cc in range(N_CH):
        k = cc % NBUF
        nk = (cc + 1) % NBUF
        if cc + 1 < N_CH:
            if cc + 1 >= NBUF:
                o_h[cc + 1 - NBUF].wait()
            build_idx(cc + 1, nk)
            g_h[cc + 1] = gather(cc + 1, nk)
        g_h[cc].wait()
        o_h[cc] = write_out(cc, k)
    o_h[N_CH - 2].wait()
    o_h[N_CH - 1].wait()


def kernel(tokens, tables):
    f = tables.shape[0]
    d = tables.shape[-1]
    tt1d = tables.transpose(0, 2, 1).reshape(-1)
    tok_flat = tokens.reshape(-1).astype(jnp.int32)
    out = _sc_gather(tt1d, tok_flat)
    return out.reshape(tokens.shape[0], f, d)
